# baseline (speedup 1.0000x reference)
import functools

import jax
import jax.numpy as jnp
from jax import lax
from jax.experimental import pallas as pl
from jax.experimental.pallas import tpu as pltpu
from jax.experimental.pallas import tpu_sc as plsc

_N = 10000
_E = 320000
_D = 128
_H = 16
_C = 3

_NC = 2
_NS = 16
_NT = _NC * _NS
_B = 128
_NB0 = 64
_NB1 = 96
_NBMAX = max(_NB0, _NB1)
_CHUNKS = _NS * (_NB0 + _NB1)
_EP = _CHUNKS * _B
_NPAD = 10112
_RPT = _NPAD // _NS
_NBUF = 8
_PF = 4
_SLAG = 4

_BLK = 5000
_GRID = _N // _BLK


def _copy_chunks(flat_hbm, v_ref, cid, sid):
  @pl.when(cid == 0)
  def _():
    pltpu.sync_copy(flat_hbm.at[pl.ds(sid * _NB0, _NB0)],
                    v_ref.at[pl.ds(0, _NB0)])

  @pl.when(cid == 1)
  def _():
    pltpu.sync_copy(flat_hbm.at[pl.ds(_NS * _NB0 + sid * _NB1, _NB1)],
                    v_ref.at[pl.ds(0, _NB1)])


def _deg_body(dst_hbm, ones_hbm, zero_hbm, out_hbm, dst_v, ones_v, acc_sh, sem):
  cid = lax.axis_index("c")
  sid = lax.axis_index("s")
  nb = jnp.where(cid == 0, _NB0, _NB1)
  _copy_chunks(dst_hbm, dst_v, cid, sid)
  pltpu.sync_copy(ones_hbm, ones_v)
  pltpu.sync_copy(zero_hbm.at[pl.ds(sid * _RPT, _RPT)],
                  acc_sh.at[pl.ds(sid * _RPT, _RPT)])
  plsc.subcore_barrier()

  def body(j, carry):
    @pl.when(j >= _SLAG)
    def _():
      pltpu.make_async_copy(ones_v, acc_sh.at[dst_v.at[j - _SLAG]], sem).wait()

    pltpu.async_copy(ones_v, acc_sh.at[dst_v.at[j]], sem, add=True)
    return carry

  lax.fori_loop(0, nb, body, 0)

  def drain(j, carry):
    pltpu.make_async_copy(ones_v, acc_sh.at[dst_v.at[j]], sem).wait()
    return carry

  lax.fori_loop(nb - _SLAG, nb, drain, 0)
  plsc.subcore_barrier()
  pltpu.sync_copy(acc_sh.at[pl.ds(sid * _RPT, _RPT)],
                  out_hbm.at[cid, pl.ds(sid * _RPT, _RPT)])


@functools.cache
def _deg_kernel():
  return functools.partial(
      pl.kernel,
      out_type=jax.ShapeDtypeStruct((_NC, _NPAD, 8), jnp.float32),
      mesh=plsc.VectorSubcoreMesh(core_axis_name="c", subcore_axis_name="s"),
      scratch_types=[
          pltpu.VMEM((_NBMAX, _B), jnp.int32),
          pltpu.VMEM((_B, 8), jnp.float32),
          pltpu.VMEM_SHARED((_NPAD, 8), jnp.float32),
          pltpu.SemaphoreType.DMA,
      ],
      compiler_params=pltpu.CompilerParams(use_tc_tiling_on_sc=False),
  )(_deg_body)


def _agg_body(hp_hbm, src_hbm, dst_hbm, zero_hbm, out_hbm,
              src_v, dst_v, rows_v, acc_sh, sem_g, sem_s):
  cid = lax.axis_index("c")
  sid = lax.axis_index("s")
  nb = jnp.where(cid == 0, _NB0, _NB1)
  _copy_chunks(src_hbm, src_v, cid, sid)
  _copy_chunks(dst_hbm, dst_v, cid, sid)
  pltpu.sync_copy(zero_hbm.at[pl.ds(sid * _RPT, _RPT)],
                  acc_sh.at[pl.ds(sid * _RPT, _RPT)])
  plsc.subcore_barrier()

  for b in range(_PF):
    pltpu.async_copy(hp_hbm.at[src_v.at[b]], rows_v.at[b], sem_g)

  def body(j, carry):
    b = lax.rem(j, _NBUF)

    @pl.when(j >= _SLAG)
    def _():
      jd = j - _SLAG
      pltpu.make_async_copy(rows_v.at[lax.rem(jd, _NBUF)],
                            acc_sh.at[dst_v.at[jd]], sem_s).wait()

    pltpu.make_async_copy(hp_hbm.at[src_v.at[j]], rows_v.at[b], sem_g).wait()
    pltpu.async_copy(rows_v.at[b], acc_sh.at[dst_v.at[j]], sem_s, add=True)

    @pl.when(j + _PF < nb)
    def _():
      pltpu.async_copy(hp_hbm.at[src_v.at[j + _PF]],
                       rows_v.at[lax.rem(j + _PF, _NBUF)], sem_g)

    return carry

  lax.fori_loop(0, nb, body, 0)

  def drain(j, carry):
    pltpu.make_async_copy(rows_v.at[lax.rem(j, _NBUF)],
                          acc_sh.at[dst_v.at[j]], sem_s).wait()
    return carry

  lax.fori_loop(nb - _SLAG, nb, drain, 0)
  plsc.subcore_barrier()
  pltpu.sync_copy(acc_sh.at[pl.ds(sid * _RPT, _RPT)],
                  out_hbm.at[cid, pl.ds(sid * _RPT, _RPT)])


@functools.cache
def _agg_kernel(width):
  return functools.partial(
      pl.kernel,
      out_type=jax.ShapeDtypeStruct((_NC, _NPAD, width), jnp.float32),
      mesh=plsc.VectorSubcoreMesh(core_axis_name="c", subcore_axis_name="s"),
      scratch_types=[
          pltpu.VMEM((_NBMAX, _B), jnp.int32),
          pltpu.VMEM((_NBMAX, _B), jnp.int32),
          pltpu.VMEM((_NBUF, _B, width), jnp.float32),
          pltpu.VMEM_SHARED((_NPAD, width), jnp.float32),
          pltpu.SemaphoreType.DMA,
          pltpu.SemaphoreType.DMA,
      ],
      compiler_params=pltpu.CompilerParams(use_tc_tiling_on_sc=False),
  )(_agg_body)


def _dis_of(d_ref):
  deg = 1.0 + d_ref[0, :, :1] + d_ref[1, :, :1]
  return lax.rsqrt(deg)


def _fwd1_body(x_ref, w_ref, d_ref, o_ref):
  h1 = jnp.dot(x_ref[...], w_ref[...], preferred_element_type=jnp.float32)
  o_ref[...] = h1 * _dis_of(d_ref)


def _layer1_body(a_ref, hp_ref, d_ref, b1_ref, w2_ref, o_ref):
  dis = _dis_of(d_ref)
  out1 = dis * (a_ref[0] + a_ref[1] + hp_ref[...]) + b1_ref[...]
  g = dis * jnp.maximum(out1, 0.0)
  o_ref[...] = jnp.dot(g, w2_ref[...], preferred_element_type=jnp.float32)


def _layer2_body(c_ref, zp_ref, d_ref, b2_ref, o_ref):
  out2 = _dis_of(d_ref) * (c_ref[0] + c_ref[1] + zp_ref[...]) + b2_ref[...]
  mask = lax.broadcasted_iota(jnp.int32, (_BLK, 8), 1) < _C
  neg = jnp.float32(-1e30)
  masked = jnp.where(mask, out2, neg)
  m = jnp.max(masked, axis=1, keepdims=True)
  e = jnp.where(mask, jnp.exp(masked - m), 0.0)
  s = jnp.log(jnp.sum(e, axis=1, keepdims=True))
  o_ref[...] = out2 - m - s


def _row_spec(width):
  return pl.BlockSpec((_BLK, width), lambda i: (i, 0))


def _part_spec(width):
  return pl.BlockSpec((_NC, _BLK, width), lambda i: (0, i, 0))


def _full_spec(shape):
  return pl.BlockSpec(shape, lambda i: tuple(0 for _ in shape))


_fwd1 = pl.pallas_call(
    _fwd1_body,
    grid=(_GRID,),
    in_specs=[_row_spec(_D), _full_spec((_D, _H)), _part_spec(8)],
    out_specs=_row_spec(_H),
    out_shape=jax.ShapeDtypeStruct((_N, _H), jnp.float32),
)

_layer1 = pl.pallas_call(
    _layer1_body,
    grid=(_GRID,),
    in_specs=[_part_spec(_H), _row_spec(_H), _part_spec(8),
              _full_spec((1, _H)), _full_spec((_H, 8))],
    out_specs=_row_spec(8),
    out_shape=jax.ShapeDtypeStruct((_N, 8), jnp.float32),
)

_layer2 = pl.pallas_call(
    _layer2_body,
    grid=(_GRID,),
    in_specs=[_part_spec(8), _row_spec(8), _part_spec(8), _full_spec((1, 8))],
    out_specs=_row_spec(8),
    out_shape=jax.ShapeDtypeStruct((_N, 8), jnp.float32),
)


@jax.jit
def kernel(x, edge_index, W1, b1, W2, b2):
  src = edge_index[0]
  dst = edge_index[1]
  pad = _EP - _E
  src_p = jnp.concatenate(
      [src, jnp.zeros((pad,), jnp.int32)]).reshape(_CHUNKS, _B)
  dst_p = jnp.concatenate(
      [dst, jnp.full((pad,), _N, jnp.int32)]).reshape(_CHUNKS, _B)

  ones8 = jnp.ones((_B, 8), jnp.float32)
  zero8 = jnp.zeros((_NPAD, 8), jnp.float32)
  zero16 = jnp.zeros((_NPAD, _H), jnp.float32)
  w2p = jnp.concatenate([W2, jnp.zeros((_H, 8 - _C), jnp.float32)], axis=1)
  b2p = jnp.concatenate([b2, jnp.zeros((8 - _C,), jnp.float32)]).reshape(1, 8)

  degp = _deg_kernel()(dst_p, ones8, zero8)
  h1p = _fwd1(x, W1, degp)
  agg1 = _agg_kernel(_H)(h1p, src_p, dst_p, zero16)
  zp = _layer1(agg1, h1p, degp, b1.reshape(1, _H), w2p)
  agg2 = _agg_kernel(8)(zp, src_p, dst_p, zero8)
  out = _layer2(agg2, zp, degp, b2p)
  return out[:, :_C]


# speedup vs baseline: 1.1003x; 1.1003x over previous
import functools

import jax
import jax.numpy as jnp
from jax import lax
from jax.experimental import pallas as pl
from jax.experimental.pallas import tpu as pltpu
from jax.experimental.pallas import tpu_sc as plsc

_N = 10000
_E = 320000
_D = 128
_H = 16
_C = 3

_NC = 2
_NS = 16
_NT = _NC * _NS
_B = 128
_NB0 = 96
_NB1 = 64
_NBMAX = max(_NB0, _NB1)
_CHUNKS = _NS * (_NB0 + _NB1)
_EP = _CHUNKS * _B
_NPAD = 10112
_RPT = _NPAD // _NS
_NBUF = 8
_PF = 4
_SLAG = 4

_BLK = 5000
_GRID = _N // _BLK


def _copy_chunks(flat_hbm, v_ref, cid, sid):
  @pl.when(cid == 0)
  def _():
    pltpu.sync_copy(flat_hbm.at[pl.ds(sid * _NB0, _NB0)],
                    v_ref.at[pl.ds(0, _NB0)])

  @pl.when(cid == 1)
  def _():
    pltpu.sync_copy(flat_hbm.at[pl.ds(_NS * _NB0 + sid * _NB1, _NB1)],
                    v_ref.at[pl.ds(0, _NB1)])


def _deg_body(dst_hbm, ones_hbm, zero_hbm, out_hbm, dst_v, ones_v, acc_sh, sem):
  cid = lax.axis_index("c")
  sid = lax.axis_index("s")
  nb = jnp.where(cid == 0, _NB0, _NB1)
  _copy_chunks(dst_hbm, dst_v, cid, sid)
  pltpu.sync_copy(ones_hbm, ones_v)
  pltpu.sync_copy(zero_hbm.at[pl.ds(sid * _RPT, _RPT)],
                  acc_sh.at[pl.ds(sid * _RPT, _RPT)])
  plsc.subcore_barrier()

  def body(j, carry):
    @pl.when(j >= _SLAG)
    def _():
      pltpu.make_async_copy(ones_v, acc_sh.at[dst_v.at[j - _SLAG]], sem).wait()

    pltpu.async_copy(ones_v, acc_sh.at[dst_v.at[j]], sem, add=True)
    return carry

  lax.fori_loop(0, nb, body, 0)

  def drain(j, carry):
    pltpu.make_async_copy(ones_v, acc_sh.at[dst_v.at[j]], sem).wait()
    return carry

  lax.fori_loop(nb - _SLAG, nb, drain, 0)
  plsc.subcore_barrier()
  pltpu.sync_copy(acc_sh.at[pl.ds(sid * _RPT, _RPT)],
                  out_hbm.at[cid, pl.ds(sid * _RPT, _RPT)])


@functools.cache
def _deg_kernel():
  return functools.partial(
      pl.kernel,
      out_type=jax.ShapeDtypeStruct((_NC, _NPAD, 8), jnp.float32),
      mesh=plsc.VectorSubcoreMesh(core_axis_name="c", subcore_axis_name="s"),
      scratch_types=[
          pltpu.VMEM((_NBMAX, _B), jnp.int32),
          pltpu.VMEM((_B, 8), jnp.float32),
          pltpu.VMEM_SHARED((_NPAD, 8), jnp.float32),
          pltpu.SemaphoreType.DMA,
      ],
      compiler_params=pltpu.CompilerParams(use_tc_tiling_on_sc=False),
  )(_deg_body)


def _agg_body(hp_hbm, src_hbm, dst_hbm, zero_hbm, out_hbm,
              src_v, dst_v, rows_v, acc_sh, sem_g, sem_s):
  cid = lax.axis_index("c")
  sid = lax.axis_index("s")
  nb = jnp.where(cid == 0, _NB0, _NB1)
  _copy_chunks(src_hbm, src_v, cid, sid)
  _copy_chunks(dst_hbm, dst_v, cid, sid)
  pltpu.sync_copy(zero_hbm.at[pl.ds(sid * _RPT, _RPT)],
                  acc_sh.at[pl.ds(sid * _RPT, _RPT)])
  plsc.subcore_barrier()

  for b in range(_PF):
    pltpu.async_copy(hp_hbm.at[src_v.at[b]], rows_v.at[b], sem_g)

  def body(j, carry):
    b = lax.rem(j, _NBUF)

    @pl.when(j >= _SLAG)
    def _():
      jd = j - _SLAG
      pltpu.make_async_copy(rows_v.at[lax.rem(jd, _NBUF)],
                            acc_sh.at[dst_v.at[jd]], sem_s).wait()

    pltpu.make_async_copy(hp_hbm.at[src_v.at[j]], rows_v.at[b], sem_g).wait()
    pltpu.async_copy(rows_v.at[b], acc_sh.at[dst_v.at[j]], sem_s, add=True)

    @pl.when(j + _PF < nb)
    def _():
      pltpu.async_copy(hp_hbm.at[src_v.at[j + _PF]],
                       rows_v.at[lax.rem(j + _PF, _NBUF)], sem_g)

    return carry

  lax.fori_loop(0, nb, body, 0)

  def drain(j, carry):
    pltpu.make_async_copy(rows_v.at[lax.rem(j, _NBUF)],
                          acc_sh.at[dst_v.at[j]], sem_s).wait()
    return carry

  lax.fori_loop(nb - _SLAG, nb, drain, 0)
  plsc.subcore_barrier()
  pltpu.sync_copy(acc_sh.at[pl.ds(sid * _RPT, _RPT)],
                  out_hbm.at[cid, pl.ds(sid * _RPT, _RPT)])


@functools.cache
def _agg_kernel(width):
  return functools.partial(
      pl.kernel,
      out_type=jax.ShapeDtypeStruct((_NC, _NPAD, width), jnp.float32),
      mesh=plsc.VectorSubcoreMesh(core_axis_name="c", subcore_axis_name="s"),
      scratch_types=[
          pltpu.VMEM((_NBMAX, _B), jnp.int32),
          pltpu.VMEM((_NBMAX, _B), jnp.int32),
          pltpu.VMEM((_NBUF, _B, width), jnp.float32),
          pltpu.VMEM_SHARED((_NPAD, width), jnp.float32),
          pltpu.SemaphoreType.DMA,
          pltpu.SemaphoreType.DMA,
      ],
      compiler_params=pltpu.CompilerParams(use_tc_tiling_on_sc=False),
  )(_agg_body)


def _dis_of(d_ref):
  deg = 1.0 + d_ref[0, :, :1] + d_ref[1, :, :1]
  return lax.rsqrt(deg)


def _fwd1_body(x_ref, w_ref, d_ref, o_ref):
  h1 = jnp.dot(x_ref[...], w_ref[...], preferred_element_type=jnp.float32)
  o_ref[...] = h1 * _dis_of(d_ref)


def _layer1_body(a_ref, hp_ref, d_ref, b1_ref, w2_ref, o_ref):
  dis = _dis_of(d_ref)
  out1 = dis * (a_ref[0] + a_ref[1] + hp_ref[...]) + b1_ref[...]
  g = dis * jnp.maximum(out1, 0.0)
  o_ref[...] = jnp.dot(g, w2_ref[...], preferred_element_type=jnp.float32)


def _layer2_body(c_ref, zp_ref, d_ref, b2_ref, o_ref):
  out2 = _dis_of(d_ref) * (c_ref[0] + c_ref[1] + zp_ref[...]) + b2_ref[...]
  mask = lax.broadcasted_iota(jnp.int32, (_BLK, 8), 1) < _C
  neg = jnp.float32(-1e30)
  masked = jnp.where(mask, out2, neg)
  m = jnp.max(masked, axis=1, keepdims=True)
  e = jnp.where(mask, jnp.exp(masked - m), 0.0)
  s = jnp.log(jnp.sum(e, axis=1, keepdims=True))
  o_ref[...] = out2 - m - s


def _row_spec(width):
  return pl.BlockSpec((_BLK, width), lambda i: (i, 0))


def _part_spec(width):
  return pl.BlockSpec((_NC, _BLK, width), lambda i: (0, i, 0))


def _full_spec(shape):
  return pl.BlockSpec(shape, lambda i: tuple(0 for _ in shape))


_fwd1 = pl.pallas_call(
    _fwd1_body,
    grid=(_GRID,),
    in_specs=[_row_spec(_D), _full_spec((_D, _H)), _part_spec(8)],
    out_specs=_row_spec(_H),
    out_shape=jax.ShapeDtypeStruct((_N, _H), jnp.float32),
)

_layer1 = pl.pallas_call(
    _layer1_body,
    grid=(_GRID,),
    in_specs=[_part_spec(_H), _row_spec(_H), _part_spec(8),
              _full_spec((1, _H)), _full_spec((_H, 8))],
    out_specs=_row_spec(8),
    out_shape=jax.ShapeDtypeStruct((_N, 8), jnp.float32),
)

_layer2 = pl.pallas_call(
    _layer2_body,
    grid=(_GRID,),
    in_specs=[_part_spec(8), _row_spec(8), _part_spec(8), _full_spec((1, 8))],
    out_specs=_row_spec(8),
    out_shape=jax.ShapeDtypeStruct((_N, 8), jnp.float32),
)


@jax.jit
def kernel(x, edge_index, W1, b1, W2, b2):
  src = edge_index[0]
  dst = edge_index[1]
  pad = _EP - _E
  src_p = jnp.concatenate(
      [src, jnp.zeros((pad,), jnp.int32)]).reshape(_CHUNKS, _B)
  dst_p = jnp.concatenate(
      [dst, jnp.full((pad,), _N, jnp.int32)]).reshape(_CHUNKS, _B)

  ones8 = jnp.ones((_B, 8), jnp.float32)
  zero8 = jnp.zeros((_NPAD, 8), jnp.float32)
  zero16 = jnp.zeros((_NPAD, _H), jnp.float32)
  w2p = jnp.concatenate([W2, jnp.zeros((_H, 8 - _C), jnp.float32)], axis=1)
  b2p = jnp.concatenate([b2, jnp.zeros((8 - _C,), jnp.float32)]).reshape(1, 8)

  degp = _deg_kernel()(dst_p, ones8, zero8)
  h1p = _fwd1(x, W1, degp)
  agg1 = _agg_kernel(_H)(h1p, src_p, dst_p, zero16)
  zp = _layer1(agg1, h1p, degp, b1.reshape(1, _H), w2p)
  agg2 = _agg_kernel(8)(zp, src_p, dst_p, zero8)
  out = _layer2(agg2, zp, degp, b2p)
  return out[:, :_C]


# speedup vs baseline: 1.2348x; 1.1223x over previous
import functools

import jax
import jax.numpy as jnp
from jax import lax
from jax.experimental import pallas as pl
from jax.experimental.pallas import tpu as pltpu
from jax.experimental.pallas import tpu_sc as plsc

_N = 10000
_E = 320000
_D = 128
_H = 16
_C = 3

_NC = 2
_NS = 16
_NT = _NC * _NS
_B = 128
_NB0 = 96
_NB1 = 64
_NBMAX = max(_NB0, _NB1)
_CHUNKS = _NS * (_NB0 + _NB1)
_EP = _CHUNKS * _B
_NPAD = 10112
_RPT = _NPAD // _NS
_NBUF = 8
_PF = 4
_SLAG = 4


def _copy_chunks(flat_hbm, v_ref, cid, sid):
  @pl.when(cid == 0)
  def _():
    pltpu.sync_copy(flat_hbm.at[pl.ds(sid * _NB0, _NB0)],
                    v_ref.at[pl.ds(0, _NB0)])

  @pl.when(cid == 1)
  def _():
    pltpu.sync_copy(flat_hbm.at[pl.ds(_NS * _NB0 + sid * _NB1, _NB1)],
                    v_ref.at[pl.ds(0, _NB1)])


def _deg_body(dst_hbm, ones_hbm, zero_hbm, out_hbm, dst_v, ones_v, acc_sh, sem):
  cid = lax.axis_index("c")
  sid = lax.axis_index("s")
  nb = jnp.where(cid == 0, _NB0, _NB1)
  _copy_chunks(dst_hbm, dst_v, cid, sid)
  pltpu.sync_copy(ones_hbm, ones_v)
  pltpu.sync_copy(zero_hbm.at[pl.ds(sid * _RPT, _RPT)],
                  acc_sh.at[pl.ds(sid * _RPT, _RPT)])
  plsc.subcore_barrier()

  def body(j, carry):
    @pl.when(j >= _SLAG)
    def _():
      pltpu.make_async_copy(ones_v, acc_sh.at[dst_v.at[j - _SLAG]], sem).wait()

    pltpu.async_copy(ones_v, acc_sh.at[dst_v.at[j]], sem, add=True)
    return carry

  lax.fori_loop(0, nb, body, 0)

  def drain(j, carry):
    pltpu.make_async_copy(ones_v, acc_sh.at[dst_v.at[j]], sem).wait()
    return carry

  lax.fori_loop(nb - _SLAG, nb, drain, 0)
  plsc.subcore_barrier()
  pltpu.sync_copy(acc_sh.at[pl.ds(sid * _RPT, _RPT)],
                  out_hbm.at[cid, pl.ds(sid * _RPT, _RPT)])


@functools.cache
def _deg_kernel():
  return functools.partial(
      pl.kernel,
      out_type=jax.ShapeDtypeStruct((_NC, _NPAD, _H), jnp.float32),
      mesh=plsc.VectorSubcoreMesh(core_axis_name="c", subcore_axis_name="s"),
      scratch_types=[
          pltpu.VMEM((_NBMAX, _B), jnp.int32),
          pltpu.VMEM((_B, _H), jnp.float32),
          pltpu.VMEM_SHARED((_NPAD, _H), jnp.float32),
          pltpu.SemaphoreType.DMA,
      ],
      compiler_params=pltpu.CompilerParams(use_tc_tiling_on_sc=False),
  )(_deg_body)


def _agg_body(hp_hbm, src_hbm, dst_hbm, zero_hbm, out_hbm,
              src_v, dst_v, rows_v, acc_sh, sem_g, sem_s):
  cid = lax.axis_index("c")
  sid = lax.axis_index("s")
  nb = jnp.where(cid == 0, _NB0, _NB1)
  _copy_chunks(src_hbm, src_v, cid, sid)
  _copy_chunks(dst_hbm, dst_v, cid, sid)
  pltpu.sync_copy(zero_hbm.at[pl.ds(sid * _RPT, _RPT)],
                  acc_sh.at[pl.ds(sid * _RPT, _RPT)])
  plsc.subcore_barrier()

  for b in range(_PF):
    pltpu.async_copy(hp_hbm.at[src_v.at[b]], rows_v.at[b], sem_g)

  def body(j, carry):
    b = lax.rem(j, _NBUF)

    @pl.when(j >= _SLAG)
    def _():
      jd = j - _SLAG
      pltpu.make_async_copy(rows_v.at[lax.rem(jd, _NBUF)],
                            acc_sh.at[dst_v.at[jd]], sem_s).wait()

    pltpu.make_async_copy(hp_hbm.at[src_v.at[j]], rows_v.at[b], sem_g).wait()
    pltpu.async_copy(rows_v.at[b], acc_sh.at[dst_v.at[j]], sem_s, add=True)

    @pl.when(j + _PF < nb)
    def _():
      pltpu.async_copy(hp_hbm.at[src_v.at[j + _PF]],
                       rows_v.at[lax.rem(j + _PF, _NBUF)], sem_g)

    return carry

  lax.fori_loop(0, nb, body, 0)

  def drain(j, carry):
    pltpu.make_async_copy(rows_v.at[lax.rem(j, _NBUF)],
                          acc_sh.at[dst_v.at[j]], sem_s).wait()
    return carry

  lax.fori_loop(nb - _SLAG, nb, drain, 0)
  plsc.subcore_barrier()
  pltpu.sync_copy(acc_sh.at[pl.ds(sid * _RPT, _RPT)],
                  out_hbm.at[cid, pl.ds(sid * _RPT, _RPT)])


@functools.cache
def _agg_kernel(width):
  return functools.partial(
      pl.kernel,
      out_type=jax.ShapeDtypeStruct((_NC, _NPAD, width), jnp.float32),
      mesh=plsc.VectorSubcoreMesh(core_axis_name="c", subcore_axis_name="s"),
      scratch_types=[
          pltpu.VMEM((_NBMAX, _B), jnp.int32),
          pltpu.VMEM((_NBMAX, _B), jnp.int32),
          pltpu.VMEM((_NBUF, _B, width), jnp.float32),
          pltpu.VMEM_SHARED((_NPAD, width), jnp.float32),
          pltpu.SemaphoreType.DMA,
          pltpu.SemaphoreType.DMA,
      ],
      compiler_params=pltpu.CompilerParams(use_tc_tiling_on_sc=False),
  )(_agg_body)


_PR = _NPAD * _H // 128
_XR = _N * _H // 128


def _dis_of(d_ref):
  return lax.rsqrt(1.0 + d_ref[0] + d_ref[1])


def _fwd1_body(x_ref, w_ref, d_ref, o_ref):
  xb = x_ref[...]
  h1 = jnp.concatenate(
      [jnp.dot(xb[:, k, :], w_ref[...], preferred_element_type=jnp.float32)
       for k in range(8)], axis=1)
  h1p = h1 * _dis_of(d_ref)[:_XR]
  o_ref[...] = jnp.concatenate(
      [h1p, jnp.zeros((_PR - _XR, 128), jnp.float32)], axis=0)


def _layer1_body(a_ref, hp_ref, d_ref, b1_ref, w2_ref, o_ref):
  dis = _dis_of(d_ref)
  out1 = dis * (a_ref[0] + a_ref[1] + hp_ref[...]) + b1_ref[...]
  g = dis * jnp.maximum(out1, 0.0)
  o_ref[...] = jnp.concatenate(
      [jnp.dot(g[:, 16 * k:16 * k + 16], w2_ref[...],
               preferred_element_type=jnp.float32)
       for k in range(8)], axis=1)


def _layer2_body(c_ref, zp_ref, d_ref, b2_ref, o_ref):
  out2 = _dis_of(d_ref) * (c_ref[0] + c_ref[1] + zp_ref[...]) + b2_ref[...]
  lane = lax.broadcasted_iota(jnp.int32, (_PR, 128), 1)
  mask3 = (lane % 16) < _C
  r1 = pltpu.roll(out2, 127, axis=1)
  r2 = pltpu.roll(out2, 126, axis=1)
  ms = jnp.maximum(jnp.maximum(out2, r1), r2)
  ms0 = jnp.where((lane % 16) == 0, ms, 0.0)
  ri = lax.broadcasted_iota(jnp.int32, (128, 128), 0)
  cj = lax.broadcasted_iota(jnp.int32, (128, 128), 1)
  gb = (ri == (cj // 16) * 16).astype(jnp.float32)
  gs = ((ri // 16) == (cj // 16)).astype(jnp.float32)
  mz = jnp.dot(ms0, gb, preferred_element_type=jnp.float32)
  e = jnp.where(mask3, jnp.exp(out2 - mz), 0.0)
  s = jnp.dot(e, gs, preferred_element_type=jnp.float32)
  o_ref[...] = out2 - mz - jnp.log(s)


_fwd1 = pl.pallas_call(
    _fwd1_body,
    out_shape=jax.ShapeDtypeStruct((_PR, 128), jnp.float32),
)

_layer1 = pl.pallas_call(
    _layer1_body,
    out_shape=jax.ShapeDtypeStruct((_PR, 128), jnp.float32),
)

_layer2 = pl.pallas_call(
    _layer2_body,
    out_shape=jax.ShapeDtypeStruct((_PR, 128), jnp.float32),
)


@jax.jit
def kernel(x, edge_index, W1, b1, W2, b2):
  src = edge_index[0]
  dst = edge_index[1]
  pad = _EP - _E
  src_p = jnp.concatenate(
      [src, jnp.zeros((pad,), jnp.int32)]).reshape(_CHUNKS, _B)
  dst_p = jnp.concatenate(
      [dst, jnp.full((pad,), _N, jnp.int32)]).reshape(_CHUNKS, _B)

  ones16 = jnp.ones((_B, _H), jnp.float32)
  zero16 = jnp.zeros((_NPAD, _H), jnp.float32)
  w2p = jnp.concatenate([W2, jnp.zeros((_H, _H - _C), jnp.float32)], axis=1)
  b1t = jnp.tile(b1, 8).reshape(1, 128)
  b2t = jnp.tile(jnp.concatenate([b2, jnp.zeros((_H - _C,), jnp.float32)]),
                 8).reshape(1, 128)
  x3 = x.reshape(_XR, 8, _D)

  degp = _deg_kernel()(dst_p, ones16, zero16)
  dp = degp.reshape(_NC, _PR, 128)
  h1p = _fwd1(x3, W1, dp)
  agg1 = _agg_kernel(_H)(h1p.reshape(_NPAD, _H), src_p, dst_p, zero16)
  zp = _layer1(agg1.reshape(_NC, _PR, 128), h1p, dp, b1t, w2p)
  agg2 = _agg_kernel(_H)(zp.reshape(_NPAD, _H), src_p, dst_p, zero16)
  out = _layer2(agg2.reshape(_NC, _PR, 128), zp, dp, b2t)
  return out.reshape(_NPAD, _H)[:_N, :_C]


# speedup vs baseline: 1.9253x; 1.5592x over previous
import functools

import jax
import jax.numpy as jnp
from jax import lax
from jax.experimental import pallas as pl
from jax.experimental.pallas import tpu as pltpu
from jax.experimental.pallas import tpu_sc as plsc

_N = 10000
_E = 320000
_D = 128
_H = 16
_C = 3

_NC = 2
_NS = 16
_NT = _NC * _NS
_B = 128
_CHUNKS = _E // _B
_CPC = _CHUNKS // _NC
_NBMAX = 79
_NPAD = 10112
_RPT = _NPAD // _NS
_NBUF = 8
_PF = 4
_SLAG = 4


def _copy_chunks(ei_hbm, which, v_ref, cid, sid):
  base = cid * _CPC

  @pl.when(sid < 2)
  def _():
    pltpu.sync_copy(ei_hbm.at[pl.ds(base + sid * 79, 79), which],
                    v_ref.at[pl.ds(0, 79)])

  @pl.when(sid >= 2)
  def _():
    pltpu.sync_copy(ei_hbm.at[pl.ds(base + sid * 78 + 2, 78), which],
                    v_ref.at[pl.ds(0, 78)])


def _deg_body(ei_hbm, ones_hbm, zero_hbm, out_hbm, dst_v, ones_v, acc_sh, sem):
  cid = lax.axis_index("c")
  sid = lax.axis_index("s")
  nb = jnp.where(sid < 2, 79, 78)
  _copy_chunks(ei_hbm, 1, dst_v, cid, sid)
  pltpu.sync_copy(ones_hbm, ones_v)
  pltpu.sync_copy(zero_hbm.at[pl.ds(sid * _RPT, _RPT)],
                  acc_sh.at[pl.ds(sid * _RPT, _RPT)])
  plsc.subcore_barrier()

  def body(j, carry):
    @pl.when(j >= _SLAG)
    def _():
      pltpu.make_async_copy(ones_v, acc_sh.at[dst_v.at[j - _SLAG]], sem).wait()

    pltpu.async_copy(ones_v, acc_sh.at[dst_v.at[j]], sem, add=True)
    return carry

  lax.fori_loop(0, nb, body, 0)

  def drain(j, carry):
    pltpu.make_async_copy(ones_v, acc_sh.at[dst_v.at[j]], sem).wait()
    return carry

  lax.fori_loop(nb - _SLAG, nb, drain, 0)
  plsc.subcore_barrier()
  pltpu.sync_copy(acc_sh.at[pl.ds(sid * _RPT, _RPT)],
                  out_hbm.at[cid, pl.ds(sid * _RPT, _RPT)])


@functools.cache
def _deg_kernel():
  return functools.partial(
      pl.kernel,
      out_type=jax.ShapeDtypeStruct((_NC, _NPAD, _H), jnp.float32),
      mesh=plsc.VectorSubcoreMesh(core_axis_name="c", subcore_axis_name="s"),
      scratch_types=[
          pltpu.VMEM((_NBMAX, _B), jnp.int32),
          pltpu.VMEM((_B, _H), jnp.float32),
          pltpu.VMEM_SHARED((_NPAD, _H), jnp.float32),
          pltpu.SemaphoreType.DMA,
      ],
      compiler_params=pltpu.CompilerParams(use_tc_tiling_on_sc=False),
  )(_deg_body)


def _agg_body(hp_hbm, ei_hbm, zero_hbm, out_hbm,
              src_v, dst_v, rows_v, acc_sh, sem_g, sem_s):
  cid = lax.axis_index("c")
  sid = lax.axis_index("s")
  nb = jnp.where(sid < 2, 79, 78)
  _copy_chunks(ei_hbm, 0, src_v, cid, sid)
  _copy_chunks(ei_hbm, 1, dst_v, cid, sid)
  pltpu.sync_copy(zero_hbm.at[pl.ds(sid * _RPT, _RPT)],
                  acc_sh.at[pl.ds(sid * _RPT, _RPT)])
  plsc.subcore_barrier()

  for b in range(_PF):
    pltpu.async_copy(hp_hbm.at[src_v.at[b]], rows_v.at[b], sem_g)

  def body(j, carry):
    b = lax.rem(j, _NBUF)

    @pl.when(j >= _SLAG)
    def _():
      jd = j - _SLAG
      pltpu.make_async_copy(rows_v.at[lax.rem(jd, _NBUF)],
                            acc_sh.at[dst_v.at[jd]], sem_s).wait()

    pltpu.make_async_copy(hp_hbm.at[src_v.at[j]], rows_v.at[b], sem_g).wait()
    pltpu.async_copy(rows_v.at[b], acc_sh.at[dst_v.at[j]], sem_s, add=True)

    @pl.when(j + _PF < nb)
    def _():
      pltpu.async_copy(hp_hbm.at[src_v.at[j + _PF]],
                       rows_v.at[lax.rem(j + _PF, _NBUF)], sem_g)

    return carry

  lax.fori_loop(0, nb, body, 0)

  def drain(j, carry):
    pltpu.make_async_copy(rows_v.at[lax.rem(j, _NBUF)],
                          acc_sh.at[dst_v.at[j]], sem_s).wait()
    return carry

  lax.fori_loop(nb - _SLAG, nb, drain, 0)
  plsc.subcore_barrier()
  pltpu.sync_copy(acc_sh.at[pl.ds(sid * _RPT, _RPT)],
                  out_hbm.at[cid, pl.ds(sid * _RPT, _RPT)])


@functools.cache
def _agg_kernel(width):
  return functools.partial(
      pl.kernel,
      out_type=jax.ShapeDtypeStruct((_NC, _NPAD, width), jnp.float32),
      mesh=plsc.VectorSubcoreMesh(core_axis_name="c", subcore_axis_name="s"),
      scratch_types=[
          pltpu.VMEM((_NBMAX, _B), jnp.int32),
          pltpu.VMEM((_NBMAX, _B), jnp.int32),
          pltpu.VMEM((_NBUF, _B, width), jnp.float32),
          pltpu.VMEM_SHARED((_NPAD, width), jnp.float32),
          pltpu.SemaphoreType.DMA,
          pltpu.SemaphoreType.DMA,
      ],
      compiler_params=pltpu.CompilerParams(use_tc_tiling_on_sc=False),
  )(_agg_body)


_PR = _NPAD * _H // 128
_XR = _N * _H // 128


def _dis_of(d_ref):
  return lax.rsqrt(1.0 + d_ref[0] + d_ref[1])


def _fwd1_body(x_ref, w_ref, d_ref, o_ref):
  xb = x_ref[...]
  h1 = jnp.concatenate(
      [jnp.dot(xb[:, k, :], w_ref[...], preferred_element_type=jnp.float32)
       for k in range(8)], axis=1)
  h1p = h1 * _dis_of(d_ref)[:_XR]
  o_ref[...] = jnp.concatenate(
      [h1p, jnp.zeros((_PR - _XR, 128), jnp.float32)], axis=0)


def _layer1_body(a_ref, hp_ref, d_ref, b1_ref, w2_ref, o_ref):
  dis = _dis_of(d_ref)
  out1 = dis * (a_ref[0] + a_ref[1] + hp_ref[...]) + b1_ref[...]
  g = dis * jnp.maximum(out1, 0.0)
  o_ref[...] = jnp.concatenate(
      [jnp.dot(g[:, 16 * k:16 * k + 16], w2_ref[...],
               preferred_element_type=jnp.float32)
       for k in range(8)], axis=1)


def _layer2_body(c_ref, zp_ref, d_ref, b2_ref, o_ref):
  out2 = _dis_of(d_ref) * (c_ref[0] + c_ref[1] + zp_ref[...]) + b2_ref[...]
  lane = lax.broadcasted_iota(jnp.int32, (_PR, 128), 1)
  mask3 = (lane % 16) < _C
  r1 = pltpu.roll(out2, 127, axis=1)
  r2 = pltpu.roll(out2, 126, axis=1)
  ms = jnp.maximum(jnp.maximum(out2, r1), r2)
  ms0 = jnp.where((lane % 16) == 0, ms, 0.0)
  ri = lax.broadcasted_iota(jnp.int32, (128, 128), 0)
  cj = lax.broadcasted_iota(jnp.int32, (128, 128), 1)
  gb = (ri == (cj // 16) * 16).astype(jnp.float32)
  gs = ((ri // 16) == (cj // 16)).astype(jnp.float32)
  mz = jnp.dot(ms0, gb, preferred_element_type=jnp.float32)
  e = jnp.where(mask3, jnp.exp(out2 - mz), 0.0)
  s = jnp.dot(e, gs, preferred_element_type=jnp.float32)
  o_ref[...] = out2 - mz - jnp.log(s)


_fwd1 = pl.pallas_call(
    _fwd1_body,
    out_shape=jax.ShapeDtypeStruct((_PR, 128), jnp.float32),
)

_layer1 = pl.pallas_call(
    _layer1_body,
    out_shape=jax.ShapeDtypeStruct((_PR, 128), jnp.float32),
)

_layer2 = pl.pallas_call(
    _layer2_body,
    out_shape=jax.ShapeDtypeStruct((_PR, 128), jnp.float32),
)


@jax.jit
def kernel(x, edge_index, W1, b1, W2, b2):
  ei3 = jnp.transpose(edge_index.reshape(2, _CHUNKS, _B), (1, 0, 2))

  ones16 = jnp.ones((_B, _H), jnp.float32)
  zero16 = jnp.zeros((_NPAD, _H), jnp.float32)
  w2p = jnp.concatenate([W2, jnp.zeros((_H, _H - _C), jnp.float32)], axis=1)
  b1t = jnp.tile(b1, 8).reshape(1, 128)
  b2t = jnp.tile(jnp.concatenate([b2, jnp.zeros((_H - _C,), jnp.float32)]),
                 8).reshape(1, 128)
  x3 = x.reshape(_XR, 8, _D)

  degp = _deg_kernel()(ei3, ones16, zero16)
  dp = degp.reshape(_NC, _PR, 128)
  h1p = _fwd1(x3, W1, dp)
  agg1 = _agg_kernel(_H)(h1p.reshape(_NPAD, _H), ei3, zero16)
  zp = _layer1(agg1.reshape(_NC, _PR, 128), h1p, dp, b1t, w2p)
  agg2 = _agg_kernel(_H)(zp.reshape(_NPAD, _H), ei3, zero16)
  out = _layer2(agg2.reshape(_NC, _PR, 128), zp, dp, b2t)
  return out.reshape(_NPAD, _H)[:_N, :_C]


# speedup vs baseline: 1.9798x; 1.0283x over previous
import functools

import jax
import jax.numpy as jnp
from jax import lax
from jax.experimental import pallas as pl
from jax.experimental.pallas import tpu as pltpu
from jax.experimental.pallas import tpu_sc as plsc

_N = 10000
_E = 320000
_D = 128
_H = 16
_C = 3

_NC = 2
_NS = 16
_NT = _NC * _NS
_B = 128
_CHUNKS = _E // _B
_CPC = _CHUNKS // _NC
_NBMAX = 80
_NPAD = 10112
_RPT = _NPAD // _NS
_NBUF = 8
_PF = 4
_SLAG = 4


def _copy_chunks(ei_hbm, which, v_ref, cid, sid):
  base = cid * _CPC

  @pl.when(sid == 0)
  def _():
    pltpu.sync_copy(ei_hbm.at[pl.ds(base, 80), which],
                    v_ref.at[pl.ds(0, 80)])

  @pl.when(sid > 0)
  def _():
    pltpu.sync_copy(ei_hbm.at[pl.ds(base + sid * 78 + 2, 78), which],
                    v_ref.at[pl.ds(0, 78)])


def _deg_body(ei_hbm, ones_hbm, zero_hbm, out_hbm, dst_v, ones_v, acc_sh, sem):
  cid = lax.axis_index("c")
  sid = lax.axis_index("s")
  ng = jnp.where(sid == 0, 80, 78)
  _copy_chunks(ei_hbm, 1, dst_v, cid, sid)
  pltpu.sync_copy(ones_hbm, ones_v)
  pltpu.sync_copy(zero_hbm.at[pl.ds(sid * _RPT, _RPT)],
                  acc_sh.at[pl.ds(sid * _RPT, _RPT)])
  plsc.subcore_barrier()

  def body(j, carry):
    @pl.when(j >= _SLAG)
    def _():
      pltpu.make_async_copy(
          ones_v, acc_sh.at[dst_v.at[j - _SLAG]], sem).wait()

    pltpu.async_copy(ones_v, acc_sh.at[dst_v.at[j]], sem, add=True)
    return carry

  lax.fori_loop(0, ng, body, 0)

  def drain(j, carry):
    pltpu.make_async_copy(ones_v, acc_sh.at[dst_v.at[j]], sem).wait()
    return carry

  lax.fori_loop(ng - _SLAG, ng, drain, 0)
  plsc.subcore_barrier()
  pltpu.sync_copy(acc_sh.at[pl.ds(sid * _RPT, _RPT)],
                  out_hbm.at[cid, pl.ds(sid * _RPT, _RPT)])


@functools.cache
def _deg_kernel():
  return functools.partial(
      pl.kernel,
      out_type=jax.ShapeDtypeStruct((_NC, _NPAD, _H), jnp.float32),
      mesh=plsc.VectorSubcoreMesh(core_axis_name="c", subcore_axis_name="s"),
      scratch_types=[
          pltpu.VMEM((_NBMAX, _B), jnp.int32),
          pltpu.VMEM((_B, _H), jnp.float32),
          pltpu.VMEM_SHARED((_NPAD, _H), jnp.float32),
          pltpu.SemaphoreType.DMA,
      ],
      compiler_params=pltpu.CompilerParams(use_tc_tiling_on_sc=False),
  )(_deg_body)


def _agg_body(hp_hbm, ei_hbm, zero_hbm, out_hbm,
              src_v, dst_v, rows_v, acc_sh, sem_g, sem_s):
  cid = lax.axis_index("c")
  sid = lax.axis_index("s")
  ng = jnp.where(sid == 0, 80, 78)
  _copy_chunks(ei_hbm, 0, src_v, cid, sid)
  _copy_chunks(ei_hbm, 1, dst_v, cid, sid)
  pltpu.sync_copy(zero_hbm.at[pl.ds(sid * _RPT, _RPT)],
                  acc_sh.at[pl.ds(sid * _RPT, _RPT)])
  plsc.subcore_barrier()

  def _sidx(j):
    return src_v.at[j]

  def _didx(j):
    return dst_v.at[j]

  for b in range(_PF):
    pltpu.async_copy(hp_hbm.at[_sidx(b)], rows_v.at[b], sem_g)

  def body(j, carry):
    b = lax.rem(j, _NBUF)

    @pl.when(j >= _SLAG)
    def _():
      jd = j - _SLAG
      pltpu.make_async_copy(rows_v.at[lax.rem(jd, _NBUF)],
                            acc_sh.at[_didx(jd)], sem_s).wait()

    pltpu.make_async_copy(hp_hbm.at[_sidx(j)], rows_v.at[b], sem_g).wait()
    pltpu.async_copy(rows_v.at[b], acc_sh.at[_didx(j)], sem_s, add=True)

    @pl.when(j + _PF < ng)
    def _():
      pltpu.async_copy(hp_hbm.at[_sidx(j + _PF)],
                       rows_v.at[lax.rem(j + _PF, _NBUF)], sem_g)

    return carry

  lax.fori_loop(0, ng, body, 0)

  def drain(j, carry):
    pltpu.make_async_copy(rows_v.at[lax.rem(j, _NBUF)],
                          acc_sh.at[_didx(j)], sem_s).wait()
    return carry

  lax.fori_loop(ng - _SLAG, ng, drain, 0)
  plsc.subcore_barrier()
  pltpu.sync_copy(acc_sh.at[pl.ds(sid * _RPT, _RPT)],
                  out_hbm.at[cid, pl.ds(sid * _RPT, _RPT)])


@functools.cache
def _agg_kernel(width):
  return functools.partial(
      pl.kernel,
      out_type=jax.ShapeDtypeStruct((_NC, _NPAD, width), jnp.float32),
      mesh=plsc.VectorSubcoreMesh(core_axis_name="c", subcore_axis_name="s"),
      scratch_types=[
          pltpu.VMEM((_NBMAX, _B), jnp.int32),
          pltpu.VMEM((_NBMAX, _B), jnp.int32),
          pltpu.VMEM((_NBUF, _B, width), jnp.float32),
          pltpu.VMEM_SHARED((_NPAD, width), jnp.float32),
          pltpu.SemaphoreType.DMA,
          pltpu.SemaphoreType.DMA,
      ],
      compiler_params=pltpu.CompilerParams(use_tc_tiling_on_sc=False),
  )(_agg_body)


_PR = _NPAD * _H // 128
_XR = _N * _H // 128


def _dis_of(d_ref):
  return lax.rsqrt(1.0 + d_ref[0] + d_ref[1])


def _fwd1_body(x_ref, w_ref, o_ref):
  xb = x_ref[...]
  h1 = jnp.concatenate(
      [jnp.dot(xb[:, k, :], w_ref[...], preferred_element_type=jnp.float32)
       for k in range(8)], axis=1)
  o_ref[...] = jnp.concatenate(
      [h1, jnp.zeros((_PR - _XR, 128), jnp.float32)], axis=0)


def _scale_body(h_ref, d_ref, o_ref):
  o_ref[...] = h_ref[...] * _dis_of(d_ref)


def _layer1_body(a_ref, hp_ref, d_ref, b1_ref, w2_ref, o_ref):
  dis = _dis_of(d_ref)
  out1 = dis * (a_ref[0] + a_ref[1] + hp_ref[...]) + b1_ref[...]
  g = dis * jnp.maximum(out1, 0.0)
  o_ref[...] = jnp.concatenate(
      [jnp.dot(g[:, 16 * k:16 * k + 16], w2_ref[...],
               preferred_element_type=jnp.float32)
       for k in range(8)], axis=1)


def _layer2_body(c_ref, zp_ref, d_ref, b2_ref, o_ref):
  out2 = _dis_of(d_ref) * (c_ref[0] + c_ref[1] + zp_ref[...]) + b2_ref[...]
  lane = lax.broadcasted_iota(jnp.int32, (_PR, 128), 1)
  mask3 = (lane % 16) < _C
  r1 = pltpu.roll(out2, 127, axis=1)
  r2 = pltpu.roll(out2, 126, axis=1)
  ms = jnp.maximum(jnp.maximum(out2, r1), r2)
  ms0 = jnp.where((lane % 16) == 0, ms, 0.0)
  ri = lax.broadcasted_iota(jnp.int32, (128, 128), 0)
  cj = lax.broadcasted_iota(jnp.int32, (128, 128), 1)
  gb = (ri == (cj // 16) * 16).astype(jnp.float32)
  gs = ((ri // 16) == (cj // 16)).astype(jnp.float32)
  mz = jnp.dot(ms0, gb, preferred_element_type=jnp.float32)
  e = jnp.where(mask3, jnp.exp(out2 - mz), 0.0)
  s = jnp.dot(e, gs, preferred_element_type=jnp.float32)
  o_ref[...] = out2 - mz - jnp.log(s)


_fwd1 = pl.pallas_call(
    _fwd1_body,
    out_shape=jax.ShapeDtypeStruct((_PR, 128), jnp.float32),
)

_scale = pl.pallas_call(
    _scale_body,
    out_shape=jax.ShapeDtypeStruct((_PR, 128), jnp.float32),
)

_layer1 = pl.pallas_call(
    _layer1_body,
    out_shape=jax.ShapeDtypeStruct((_PR, 128), jnp.float32),
)

_layer2 = pl.pallas_call(
    _layer2_body,
    out_shape=jax.ShapeDtypeStruct((_PR, 128), jnp.float32),
)


@jax.jit
def kernel(x, edge_index, W1, b1, W2, b2):
  ei3 = jnp.transpose(edge_index.reshape(2, _CHUNKS, _B), (1, 0, 2))

  ones16 = jnp.ones((_B, _H), jnp.float32)
  zero16 = jnp.zeros((_NPAD, _H), jnp.float32)
  w2p = jnp.concatenate([W2, jnp.zeros((_H, _H - _C), jnp.float32)], axis=1)
  b1t = jnp.tile(b1, 8).reshape(1, 128)
  b2t = jnp.tile(jnp.concatenate([b2, jnp.zeros((_H - _C,), jnp.float32)]),
                 8).reshape(1, 128)
  x3 = x.reshape(_XR, 8, _D)

  degp = _deg_kernel()(ei3, ones16, zero16)
  h1 = _fwd1(x3, W1)
  dp = degp.reshape(_NC, _PR, 128)
  h1p = _scale(h1, dp)
  agg1 = _agg_kernel(_H)(h1p.reshape(_NPAD, _H), ei3, zero16)
  zp = _layer1(agg1.reshape(_NC, _PR, 128), h1p, dp, b1t, w2p)
  agg2 = _agg_kernel(_H)(zp.reshape(_NPAD, _H), ei3, zero16)
  out = _layer2(agg2.reshape(_NC, _PR, 128), zp, dp, b2t)
  return out.reshape(_NPAD, _H)[:_N, :_C]


# speedup vs baseline: 2.4497x; 1.2374x over previous
import functools

import jax
import jax.numpy as jnp
from jax import lax
from jax.experimental import pallas as pl
from jax.experimental.pallas import tpu as pltpu
from jax.experimental.pallas import tpu_sc as plsc

_N = 10000
_E = 320000
_D = 128
_H = 16
_C = 3

_NC = 2
_NS = 16
_NT = _NC * _NS
_B = 128
_CHUNKS = _E // _B
_CPC = _CHUNKS // _NC
_NBMAX = 80
_NPAD = 10112
_RPT = _NPAD // _NS
_NBUF = 12
_PF = 8
_SLAG = 4


def _copy_chunks(ei_hbm, which, v_ref, cid, sid):
  base = cid * _CPC

  @pl.when(sid == 0)
  def _():
    pltpu.sync_copy(ei_hbm.at[pl.ds(base, 80), which],
                    v_ref.at[pl.ds(0, 80)])

  @pl.when(sid > 0)
  def _():
    pltpu.sync_copy(ei_hbm.at[pl.ds(base + sid * 78 + 2, 78), which],
                    v_ref.at[pl.ds(0, 78)])


def _deg_body(ei_hbm, ones_hbm, zero_hbm, out_hbm, dst_v, ones_v, acc_sh, sem):
  cid = lax.axis_index("c")
  sid = lax.axis_index("s")
  ng = jnp.where(sid == 0, 80, 78)
  _copy_chunks(ei_hbm, 1, dst_v, cid, sid)
  pltpu.sync_copy(ones_hbm, ones_v)
  pltpu.sync_copy(zero_hbm.at[pl.ds(sid * _RPT, _RPT)],
                  acc_sh.at[pl.ds(sid * _RPT, _RPT)])
  plsc.subcore_barrier()

  def body(j, carry):
    @pl.when(j >= _SLAG)
    def _():
      pltpu.make_async_copy(
          ones_v, acc_sh.at[dst_v.at[j - _SLAG]], sem).wait()

    pltpu.async_copy(ones_v, acc_sh.at[dst_v.at[j]], sem, add=True)
    return carry

  lax.fori_loop(0, ng, body, 0)

  def drain(j, carry):
    pltpu.make_async_copy(ones_v, acc_sh.at[dst_v.at[j]], sem).wait()
    return carry

  lax.fori_loop(ng - _SLAG, ng, drain, 0)
  plsc.subcore_barrier()
  pltpu.sync_copy(acc_sh.at[pl.ds(sid * _RPT, _RPT)],
                  out_hbm.at[cid, pl.ds(sid * _RPT, _RPT)])


@functools.cache
def _deg_kernel():
  return functools.partial(
      pl.kernel,
      out_type=jax.ShapeDtypeStruct((_NC, _NPAD, _H), jnp.float32),
      mesh=plsc.VectorSubcoreMesh(core_axis_name="c", subcore_axis_name="s"),
      scratch_types=[
          pltpu.VMEM((_NBMAX, _B), jnp.int32),
          pltpu.VMEM((_B, _H), jnp.float32),
          pltpu.VMEM_SHARED((_NPAD, _H), jnp.float32),
          pltpu.SemaphoreType.DMA,
      ],
      compiler_params=pltpu.CompilerParams(use_tc_tiling_on_sc=False),
  )(_deg_body)


def _agg_body(hp_hbm, ei_hbm, zero_hbm, out_hbm,
              src_v, dst_v, rows_v, acc_sh, sem_g, sem_s):
  cid = lax.axis_index("c")
  sid = lax.axis_index("s")
  ng = jnp.where(sid == 0, 80, 78)
  _copy_chunks(ei_hbm, 0, src_v, cid, sid)
  _copy_chunks(ei_hbm, 1, dst_v, cid, sid)
  pltpu.sync_copy(zero_hbm.at[pl.ds(sid * _RPT, _RPT)],
                  acc_sh.at[pl.ds(sid * _RPT, _RPT)])
  plsc.subcore_barrier()

  def _sidx(j):
    return src_v.at[j]

  def _didx(j):
    return dst_v.at[j]

  for b in range(_PF):
    pltpu.async_copy(hp_hbm.at[_sidx(b)], rows_v.at[b], sem_g)

  def body(j, carry):
    b = lax.rem(j, _NBUF)

    @pl.when(j >= _SLAG)
    def _():
      jd = j - _SLAG
      pltpu.make_async_copy(rows_v.at[lax.rem(jd, _NBUF)],
                            acc_sh.at[_didx(jd)], sem_s).wait()

    pltpu.make_async_copy(hp_hbm.at[_sidx(j)], rows_v.at[b], sem_g).wait()
    pltpu.async_copy(rows_v.at[b], acc_sh.at[_didx(j)], sem_s, add=True)

    @pl.when(j + _PF < ng)
    def _():
      pltpu.async_copy(hp_hbm.at[_sidx(j + _PF)],
                       rows_v.at[lax.rem(j + _PF, _NBUF)], sem_g)

    return carry

  lax.fori_loop(0, ng, body, 0)

  def drain(j, carry):
    pltpu.make_async_copy(rows_v.at[lax.rem(j, _NBUF)],
                          acc_sh.at[_didx(j)], sem_s).wait()
    return carry

  lax.fori_loop(ng - _SLAG, ng, drain, 0)
  plsc.subcore_barrier()
  pltpu.sync_copy(acc_sh.at[pl.ds(sid * _RPT, _RPT)],
                  out_hbm.at[cid, pl.ds(sid * _RPT, _RPT)])


@functools.cache
def _agg_kernel(width):
  return functools.partial(
      pl.kernel,
      out_type=jax.ShapeDtypeStruct((_NC, _NPAD, width), jnp.float32),
      mesh=plsc.VectorSubcoreMesh(core_axis_name="c", subcore_axis_name="s"),
      scratch_types=[
          pltpu.VMEM((_NBMAX, _B), jnp.int32),
          pltpu.VMEM((_NBMAX, _B), jnp.int32),
          pltpu.VMEM((_NBUF, _B, width), jnp.float32),
          pltpu.VMEM_SHARED((_NPAD, width), jnp.float32),
          pltpu.SemaphoreType.DMA,
          pltpu.SemaphoreType.DMA,
      ],
      compiler_params=pltpu.CompilerParams(use_tc_tiling_on_sc=False),
  )(_agg_body)


_PR = _NPAD * _H // 128
_XR = _N * _H // 128


def _dis_of(d_ref):
  return lax.rsqrt(1.0 + d_ref[0] + d_ref[1])


def _fwd1_body(x_ref, w_ref, o_ref):
  xb = x_ref[...]
  h1 = jnp.concatenate(
      [jnp.dot(xb[:, k, :], w_ref[...], preferred_element_type=jnp.float32)
       for k in range(8)], axis=1)
  o_ref[...] = jnp.concatenate(
      [h1, jnp.zeros((_PR - _XR, 128), jnp.float32)], axis=0)


def _scale_body(h_ref, d_ref, o_ref):
  o_ref[...] = h_ref[...] * _dis_of(d_ref)


def _layer1_body(a_ref, hp_ref, d_ref, b1_ref, w2_ref, o_ref):
  dis = _dis_of(d_ref)
  out1 = dis * (a_ref[0] + a_ref[1] + hp_ref[...]) + b1_ref[...]
  g = dis * jnp.maximum(out1, 0.0)
  o_ref[...] = jnp.concatenate(
      [jnp.dot(g[:, 16 * k:16 * k + 16], w2_ref[...],
               preferred_element_type=jnp.float32)
       for k in range(8)], axis=1)


def _layer2_body(c_ref, zp_ref, d_ref, b2_ref, o_ref):
  out2 = _dis_of(d_ref) * (c_ref[0] + c_ref[1] + zp_ref[...]) + b2_ref[...]
  lane = lax.broadcasted_iota(jnp.int32, (_PR, 128), 1)
  mask3 = (lane % 16) < _C
  r1 = pltpu.roll(out2, 127, axis=1)
  r2 = pltpu.roll(out2, 126, axis=1)
  ms = jnp.maximum(jnp.maximum(out2, r1), r2)
  ms0 = jnp.where((lane % 16) == 0, ms, 0.0)
  ri = lax.broadcasted_iota(jnp.int32, (128, 128), 0)
  cj = lax.broadcasted_iota(jnp.int32, (128, 128), 1)
  gb = (ri == (cj // 16) * 16).astype(jnp.float32)
  gs = ((ri // 16) == (cj // 16)).astype(jnp.float32)
  mz = jnp.dot(ms0, gb, preferred_element_type=jnp.float32)
  e = jnp.where(mask3, jnp.exp(out2 - mz), 0.0)
  s = jnp.dot(e, gs, preferred_element_type=jnp.float32)
  o_ref[...] = out2 - mz - jnp.log(s)


_fwd1 = pl.pallas_call(
    _fwd1_body,
    out_shape=jax.ShapeDtypeStruct((_PR, 128), jnp.float32),
)

_scale = pl.pallas_call(
    _scale_body,
    out_shape=jax.ShapeDtypeStruct((_PR, 128), jnp.float32),
)

_layer1 = pl.pallas_call(
    _layer1_body,
    out_shape=jax.ShapeDtypeStruct((_PR, 128), jnp.float32),
)

_layer2 = pl.pallas_call(
    _layer2_body,
    out_shape=jax.ShapeDtypeStruct((_PR, 128), jnp.float32),
)


@jax.jit
def kernel(x, edge_index, W1, b1, W2, b2):
  ei3 = jnp.transpose(edge_index.reshape(2, _CHUNKS, _B), (1, 0, 2))

  ones16 = jnp.ones((_B, _H), jnp.float32)
  zero16 = jnp.zeros((_NPAD, _H), jnp.float32)
  w2p = jnp.concatenate([W2, jnp.zeros((_H, _H - _C), jnp.float32)], axis=1)
  b1t = jnp.tile(b1, 8).reshape(1, 128)
  b2t = jnp.tile(jnp.concatenate([b2, jnp.zeros((_H - _C,), jnp.float32)]),
                 8).reshape(1, 128)
  x3 = x.reshape(_XR, 8, _D)

  degp = _deg_kernel()(ei3, ones16, zero16)
  h1 = _fwd1(x3, W1)
  dp = degp.reshape(_NC, _PR, 128)
  h1p = _scale(h1, dp)
  agg1 = _agg_kernel(_H)(h1p.reshape(_NPAD, _H), ei3, zero16)
  zp = _layer1(agg1.reshape(_NC, _PR, 128), h1p, dp, b1t, w2p)
  agg2 = _agg_kernel(_H)(zp.reshape(_NPAD, _H), ei3, zero16)
  out = _layer2(agg2.reshape(_NC, _PR, 128), zp, dp, b2t)
  return out.reshape(_NPAD, _H)[:_N, :_C]


# speedup vs baseline: 2.6343x; 1.0753x over previous
import functools

import jax
import jax.numpy as jnp
from jax import lax
from jax.experimental import pallas as pl
from jax.experimental.pallas import tpu as pltpu
from jax.experimental.pallas import tpu_sc as plsc

_N = 10000
_E = 320000
_D = 128
_H = 16
_C = 3

_NC = 2
_NS = 16
_NT = _NC * _NS
_B = 128
_CHUNKS = _E // _B
_CPC = _CHUNKS // _NC
_NBMAX = 80
_NPAD = 10112
_RPT = _NPAD // _NS
_NBUF = 24
_PF = 16
_SLAG = 8


def _copy_chunks(ei_hbm, which, v_ref, cid, sid):
  base = cid * _CPC

  @pl.when(sid == 0)
  def _():
    pltpu.sync_copy(ei_hbm.at[pl.ds(base, 80), which],
                    v_ref.at[pl.ds(0, 80)])

  @pl.when(sid > 0)
  def _():
    pltpu.sync_copy(ei_hbm.at[pl.ds(base + sid * 78 + 2, 78), which],
                    v_ref.at[pl.ds(0, 78)])


def _deg_body(ei_hbm, ones_hbm, zero_hbm, out_hbm, dst_v, ones_v, acc_sh, sem):
  cid = lax.axis_index("c")
  sid = lax.axis_index("s")
  ng = jnp.where(sid == 0, 80, 78)
  _copy_chunks(ei_hbm, 1, dst_v, cid, sid)
  pltpu.sync_copy(ones_hbm, ones_v)
  pltpu.sync_copy(zero_hbm.at[pl.ds(sid * _RPT, _RPT)],
                  acc_sh.at[pl.ds(sid * _RPT, _RPT)])
  plsc.subcore_barrier()

  def body(j, carry):
    @pl.when(j >= _SLAG)
    def _():
      pltpu.make_async_copy(
          ones_v, acc_sh.at[dst_v.at[j - _SLAG]], sem).wait()

    pltpu.async_copy(ones_v, acc_sh.at[dst_v.at[j]], sem, add=True)
    return carry

  lax.fori_loop(0, ng, body, 0)

  def drain(j, carry):
    pltpu.make_async_copy(ones_v, acc_sh.at[dst_v.at[j]], sem).wait()
    return carry

  lax.fori_loop(ng - _SLAG, ng, drain, 0)
  plsc.subcore_barrier()
  pltpu.sync_copy(acc_sh.at[pl.ds(sid * _RPT, _RPT)],
                  out_hbm.at[cid, pl.ds(sid * _RPT, _RPT)])


@functools.cache
def _deg_kernel():
  return functools.partial(
      pl.kernel,
      out_type=jax.ShapeDtypeStruct((_NC, _NPAD, _H), jnp.float32),
      mesh=plsc.VectorSubcoreMesh(core_axis_name="c", subcore_axis_name="s"),
      scratch_types=[
          pltpu.VMEM((_NBMAX, _B), jnp.int32),
          pltpu.VMEM((_B, _H), jnp.float32),
          pltpu.VMEM_SHARED((_NPAD, _H), jnp.float32),
          pltpu.SemaphoreType.DMA,
      ],
      compiler_params=pltpu.CompilerParams(use_tc_tiling_on_sc=False),
  )(_deg_body)


def _agg_body(hp_hbm, ei_hbm, zero_hbm, out_hbm,
              src_v, dst_v, rows_v, acc_sh, sem_g, sem_s):
  cid = lax.axis_index("c")
  sid = lax.axis_index("s")
  ng = jnp.where(sid == 0, 80, 78)
  _copy_chunks(ei_hbm, 0, src_v, cid, sid)
  _copy_chunks(ei_hbm, 1, dst_v, cid, sid)
  pltpu.sync_copy(zero_hbm.at[pl.ds(sid * _RPT, _RPT)],
                  acc_sh.at[pl.ds(sid * _RPT, _RPT)])
  plsc.subcore_barrier()

  def _sidx(j):
    return src_v.at[j]

  def _didx(j):
    return dst_v.at[j]

  for b in range(_PF):
    pltpu.async_copy(hp_hbm.at[_sidx(b)], rows_v.at[b], sem_g)

  def body(j, carry):
    b = lax.rem(j, _NBUF)

    @pl.when(j >= _SLAG)
    def _():
      jd = j - _SLAG
      pltpu.make_async_copy(rows_v.at[lax.rem(jd, _NBUF)],
                            acc_sh.at[_didx(jd)], sem_s).wait()

    pltpu.make_async_copy(hp_hbm.at[_sidx(j)], rows_v.at[b], sem_g).wait()
    pltpu.async_copy(rows_v.at[b], acc_sh.at[_didx(j)], sem_s, add=True)

    @pl.when(j + _PF < ng)
    def _():
      pltpu.async_copy(hp_hbm.at[_sidx(j + _PF)],
                       rows_v.at[lax.rem(j + _PF, _NBUF)], sem_g)

    return carry

  lax.fori_loop(0, ng, body, 0)

  def drain(j, carry):
    pltpu.make_async_copy(rows_v.at[lax.rem(j, _NBUF)],
                          acc_sh.at[_didx(j)], sem_s).wait()
    return carry

  lax.fori_loop(ng - _SLAG, ng, drain, 0)
  plsc.subcore_barrier()
  pltpu.sync_copy(acc_sh.at[pl.ds(sid * _RPT, _RPT)],
                  out_hbm.at[cid, pl.ds(sid * _RPT, _RPT)])


@functools.cache
def _agg_kernel(width):
  return functools.partial(
      pl.kernel,
      out_type=jax.ShapeDtypeStruct((_NC, _NPAD, width), jnp.float32),
      mesh=plsc.VectorSubcoreMesh(core_axis_name="c", subcore_axis_name="s"),
      scratch_types=[
          pltpu.VMEM((_NBMAX, _B), jnp.int32),
          pltpu.VMEM((_NBMAX, _B), jnp.int32),
          pltpu.VMEM((_NBUF, _B, width), jnp.float32),
          pltpu.VMEM_SHARED((_NPAD, width), jnp.float32),
          pltpu.SemaphoreType.DMA,
          pltpu.SemaphoreType.DMA,
      ],
      compiler_params=pltpu.CompilerParams(use_tc_tiling_on_sc=False),
  )(_agg_body)


_PR = _NPAD * _H // 128
_XR = _N * _H // 128


def _dis_of(d_ref):
  return lax.rsqrt(1.0 + d_ref[0] + d_ref[1])


def _fwd1_body(x_ref, w_ref, o_ref):
  xb = x_ref[...]
  h1 = jnp.concatenate(
      [jnp.dot(xb[:, k, :], w_ref[...], preferred_element_type=jnp.float32)
       for k in range(8)], axis=1)
  o_ref[...] = jnp.concatenate(
      [h1, jnp.zeros((_PR - _XR, 128), jnp.float32)], axis=0)


def _scale_body(h_ref, d_ref, o_ref):
  o_ref[...] = h_ref[...] * _dis_of(d_ref)


def _layer1_body(a_ref, hp_ref, d_ref, b1_ref, w2_ref, o_ref):
  dis = _dis_of(d_ref)
  out1 = dis * (a_ref[0] + a_ref[1] + hp_ref[...]) + b1_ref[...]
  g = dis * jnp.maximum(out1, 0.0)
  o_ref[...] = jnp.concatenate(
      [jnp.dot(g[:, 16 * k:16 * k + 16], w2_ref[...],
               preferred_element_type=jnp.float32)
       for k in range(8)], axis=1)


def _layer2_body(c_ref, zp_ref, d_ref, b2_ref, o_ref):
  out2 = _dis_of(d_ref) * (c_ref[0] + c_ref[1] + zp_ref[...]) + b2_ref[...]
  lane = lax.broadcasted_iota(jnp.int32, (_PR, 128), 1)
  mask3 = (lane % 16) < _C
  r1 = pltpu.roll(out2, 127, axis=1)
  r2 = pltpu.roll(out2, 126, axis=1)
  ms = jnp.maximum(jnp.maximum(out2, r1), r2)
  ms0 = jnp.where((lane % 16) == 0, ms, 0.0)
  ri = lax.broadcasted_iota(jnp.int32, (128, 128), 0)
  cj = lax.broadcasted_iota(jnp.int32, (128, 128), 1)
  gb = (ri == (cj // 16) * 16).astype(jnp.float32)
  gs = ((ri // 16) == (cj // 16)).astype(jnp.float32)
  mz = jnp.dot(ms0, gb, preferred_element_type=jnp.float32)
  e = jnp.where(mask3, jnp.exp(out2 - mz), 0.0)
  s = jnp.dot(e, gs, preferred_element_type=jnp.float32)
  o_ref[...] = out2 - mz - jnp.log(s)


_fwd1 = pl.pallas_call(
    _fwd1_body,
    out_shape=jax.ShapeDtypeStruct((_PR, 128), jnp.float32),
)

_scale = pl.pallas_call(
    _scale_body,
    out_shape=jax.ShapeDtypeStruct((_PR, 128), jnp.float32),
)

_layer1 = pl.pallas_call(
    _layer1_body,
    out_shape=jax.ShapeDtypeStruct((_PR, 128), jnp.float32),
)

_layer2 = pl.pallas_call(
    _layer2_body,
    out_shape=jax.ShapeDtypeStruct((_PR, 128), jnp.float32),
)


@jax.jit
def kernel(x, edge_index, W1, b1, W2, b2):
  ei3 = jnp.transpose(edge_index.reshape(2, _CHUNKS, _B), (1, 0, 2))

  ones16 = jnp.ones((_B, _H), jnp.float32)
  zero16 = jnp.zeros((_NPAD, _H), jnp.float32)
  w2p = jnp.concatenate([W2, jnp.zeros((_H, _H - _C), jnp.float32)], axis=1)
  b1t = jnp.tile(b1, 8).reshape(1, 128)
  b2t = jnp.tile(jnp.concatenate([b2, jnp.zeros((_H - _C,), jnp.float32)]),
                 8).reshape(1, 128)
  x3 = x.reshape(_XR, 8, _D)

  degp = _deg_kernel()(ei3, ones16, zero16)
  h1 = _fwd1(x3, W1)
  dp = degp.reshape(_NC, _PR, 128)
  h1p = _scale(h1, dp)
  agg1 = _agg_kernel(_H)(h1p.reshape(_NPAD, _H), ei3, zero16)
  zp = _layer1(agg1.reshape(_NC, _PR, 128), h1p, dp, b1t, w2p)
  agg2 = _agg_kernel(_H)(zp.reshape(_NPAD, _H), ei3, zero16)
  out = _layer2(agg2.reshape(_NC, _PR, 128), zp, dp, b2t)
  return out.reshape(_NPAD, _H)[:_N, :_C]
